# baseline (speedup 1.0000x reference)
import functools

import jax
import jax.numpy as jnp
from jax import lax
from jax.experimental import pallas as pl
from jax.experimental.pallas import tpu as pltpu
from jax.experimental.pallas import tpu_sc as plsc


def _make_sc_gather(d, b, s):
    info = plsc.get_sparse_core_info()
    nw = info.num_cores * info.num_subcores
    nc = info.num_cores
    n = b * s
    t_per_w = n // nw
    tc = 64
    n_chunks = t_per_w // tc
    w_per_row = s // t_per_w

    mesh = plsc.VectorSubcoreMesh(core_axis_name="c", subcore_axis_name="s")

    @functools.partial(
        pl.kernel,
        mesh=mesh,
        out_type=jax.ShapeDtypeStruct((n, d), jnp.float32),
        scratch_types=[
            pltpu.VMEM((tc,), jnp.int32),
            pltpu.VMEM((tc,), jnp.int32),
            pltpu.VMEM((tc, d), jnp.float32),
            pltpu.VMEM((tc, d), jnp.float32),
            pltpu.SemaphoreType.DMA,
            pltpu.SemaphoreType.DMA,
        ],
    )
    def gather_kernel(ids_hbm, word_hbm, out_hbm,
                      idx0, idx1, rows0, rows1, sem0, sem1):
        wid = lax.axis_index("s") * nc + lax.axis_index("c")
        row = wid // w_per_row
        s_off = (wid % w_per_row) * t_per_w
        base = wid * t_per_w
        idx = (idx0, idx1)
        rows = (rows0, rows1)
        sem = (sem0, sem1)
        pltpu.sync_copy(ids_hbm.at[row, pl.ds(s_off, tc)], idx[0])
        copies = [pltpu.async_copy(word_hbm.at[idx[0]], rows[0], sem[0])]
        for c in range(n_chunks):
            p = c % 2
            if c + 1 < n_chunks:
                pn = (c + 1) % 2
                pltpu.sync_copy(
                    ids_hbm.at[row, pl.ds(s_off + (c + 1) * tc, tc)], idx[pn])
                copies.append(
                    pltpu.async_copy(word_hbm.at[idx[pn]], rows[pn], sem[pn]))
            copies[c].wait()
            pltpu.sync_copy(rows[p], out_hbm.at[pl.ds(base + c * tc, tc)])

    return gather_kernel


def _ln_body(g_ref, p_ref, tt_ref, te_ref, gamma_ref, beta_ref, o_ref):
    g = g_ref[...]
    p = p_ref[...]
    t = tt_ref[...]
    te = te_ref[...]
    h = g + p[None] + te[0:1, :] + t * (te[1:2, :] - te[0:1, :])
    mu = jnp.mean(h, axis=-1, keepdims=True)
    var = jnp.mean((h - mu) ** 2, axis=-1, keepdims=True)
    o_ref[...] = ((h - mu) * lax.rsqrt(var + 1e-5)) * gamma_ref[...] + beta_ref[...]


def _sum_layernorm(gathered, pos_emb, tt_f, type_emb, gamma, beta, blk):
    b, s, d = gathered.shape
    grid = (s // blk,)
    return pl.pallas_call(
        _ln_body,
        grid=grid,
        in_specs=[
            pl.BlockSpec((b, blk, d), lambda i: (0, i, 0)),
            pl.BlockSpec((blk, d), lambda i: (i, 0)),
            pl.BlockSpec((b, blk, 1), lambda i: (0, i, 0)),
            pl.BlockSpec((2, d), lambda i: (0, 0)),
            pl.BlockSpec((1, d), lambda i: (0, 0)),
            pl.BlockSpec((1, d), lambda i: (0, 0)),
        ],
        out_specs=pl.BlockSpec((b, blk, d), lambda i: (0, i, 0)),
        out_shape=jax.ShapeDtypeStruct((b, s, d), jnp.float32),
    )(gathered, pos_emb, tt_f, type_emb, gamma, beta)


def kernel(input_ids, token_type_ids, word_emb, pos_emb, type_emb, gamma, beta):
    b, s = input_ids.shape
    vocab, d = word_emb.shape

    gathered = _make_sc_gather(d, b, s)(input_ids, word_emb)

    tt_f = token_type_ids.reshape(b, s, 1).astype(jnp.float32)
    out = _sum_layernorm(
        gathered.reshape(b, s, d), pos_emb, tt_f, type_emb,
        gamma.reshape(1, d), beta.reshape(1, d), blk=128,
    )
    return out


# speedup vs baseline: 1.0700x; 1.0700x over previous
import functools

import jax
import jax.numpy as jnp
from jax import lax
from jax.experimental import pallas as pl
from jax.experimental.pallas import tpu as pltpu
from jax.experimental.pallas import tpu_sc as plsc


def _make_sc_gather(d, b, s):
    info = plsc.get_sparse_core_info()
    nw = info.num_cores * info.num_subcores
    nc = info.num_cores
    n = b * s
    t_per_w = n // nw
    tc = 64
    n_chunks = t_per_w // tc
    w_per_row = s // t_per_w

    mesh = plsc.VectorSubcoreMesh(core_axis_name="c", subcore_axis_name="s")

    @functools.partial(
        pl.kernel,
        mesh=mesh,
        out_type=jax.ShapeDtypeStruct((n, d), jnp.float32),
        scratch_types=[
            pltpu.VMEM((tc,), jnp.int32),
            pltpu.VMEM((tc,), jnp.int32),
            pltpu.VMEM((tc, d), jnp.float32),
            pltpu.VMEM((tc, d), jnp.float32),
            pltpu.SemaphoreType.DMA,
            pltpu.SemaphoreType.DMA,
        ],
    )
    def gather_kernel(ids_hbm, word_hbm, out_hbm,
                      idx0, idx1, rows0, rows1, sem0, sem1):
        wid = lax.axis_index("s") * nc + lax.axis_index("c")
        row = wid // w_per_row
        s_off = (wid % w_per_row) * t_per_w
        base = wid * t_per_w
        idx = (idx0, idx1)
        rows = (rows0, rows1)
        sem = (sem0, sem1)
        pltpu.sync_copy(ids_hbm.at[row, pl.ds(s_off, tc)], idx[0])
        copies = [pltpu.async_copy(word_hbm.at[idx[0]], rows[0], sem[0])]
        for c in range(n_chunks):
            p = c % 2
            if c + 1 < n_chunks:
                pn = (c + 1) % 2
                pltpu.sync_copy(
                    ids_hbm.at[row, pl.ds(s_off + (c + 1) * tc, tc)], idx[pn])
                copies.append(
                    pltpu.async_copy(word_hbm.at[idx[pn]], rows[pn], sem[pn]))
            copies[c].wait()
            pltpu.sync_copy(rows[p], out_hbm.at[pl.ds(base + c * tc, tc)])

    return gather_kernel


def _ln_body(g_ref, p_ref, tt_ref, te_ref, gamma_ref, beta_ref, o_ref):
    g = g_ref[...]
    p = p_ref[...]
    t = tt_ref[...]
    te = te_ref[...]
    h = g + p[None] + te[0:1, :] + t * (te[1:2, :] - te[0:1, :])
    mu = jnp.mean(h, axis=-1, keepdims=True)
    var = jnp.mean((h - mu) ** 2, axis=-1, keepdims=True)
    o_ref[...] = ((h - mu) * lax.rsqrt(var + 1e-5)) * gamma_ref[...] + beta_ref[...]


def _sum_layernorm(gathered, pos_emb, tt_f, type_emb, gamma, beta, blk):
    b, s, d = gathered.shape
    grid = (s // blk,)
    return pl.pallas_call(
        _ln_body,
        grid=grid,
        in_specs=[
            pl.BlockSpec((b, blk, d), lambda i: (0, i, 0)),
            pl.BlockSpec((blk, d), lambda i: (i, 0)),
            pl.BlockSpec((b, blk, 1), lambda i: (0, i, 0)),
            pl.BlockSpec((2, d), lambda i: (0, 0)),
            pl.BlockSpec((1, d), lambda i: (0, 0)),
            pl.BlockSpec((1, d), lambda i: (0, 0)),
        ],
        out_specs=pl.BlockSpec((b, blk, d), lambda i: (0, i, 0)),
        out_shape=jax.ShapeDtypeStruct((b, s, d), jnp.float32),
    )(gathered, pos_emb, tt_f, type_emb, gamma, beta)


def kernel(input_ids, token_type_ids, word_emb, pos_emb, type_emb, gamma, beta):
    b, s = input_ids.shape
    vocab, d = word_emb.shape

    gathered = _make_sc_gather(d, b, s)(input_ids, word_emb)

    tt_f = token_type_ids.reshape(b, s, 1).astype(jnp.float32)
    out = _sum_layernorm(
        gathered.reshape(b, s, d), pos_emb, tt_f, type_emb,
        gamma.reshape(1, d), beta.reshape(1, d), blk=512,
    )
    return out
